# Initial kernel scaffold; baseline (speedup 1.0000x reference)
import jax
import jax.numpy as jnp
from jax.experimental import pallas as pl


def kernel(indices, table, W, U, b, Wfc, bfc):
    raise NotImplementedError("write your pallas kernel here")


# speedup vs baseline: 11.2882x; 11.2882x over previous
import functools

import jax
import jax.numpy as jnp
from jax import lax
from jax.experimental import pallas as pl
from jax.experimental.pallas import tpu as pltpu
from jax.experimental.pallas import tpu_sc as plsc

HP = 128
CP = 64


def _gw_body(t_ref, w_ref, b_ref, o_ref):
    o_ref[...] = (
        jnp.dot(t_ref[...], w_ref[...], preferred_element_type=jnp.float32)
        + b_ref[...]
    )


def _table_times_w(table, Wp, bp, rblk=2000):
    V, E = table.shape
    return pl.pallas_call(
        _gw_body,
        grid=(V // rblk,),
        in_specs=[
            pl.BlockSpec((rblk, E), lambda i: (i, 0)),
            pl.BlockSpec((E, HP), lambda i: (0, 0)),
            pl.BlockSpec((1, HP), lambda i: (0, 0)),
        ],
        out_specs=pl.BlockSpec((rblk, HP), lambda i: (i, 0)),
        out_shape=jax.ShapeDtypeStruct((V, HP), jnp.float32),
    )(table, Wp, bp)


def _make_gather(LB, nc, ns, k_grp=5):
    nw = nc * ns
    per_w = LB // nw
    grp = k_grp * 128
    n_groups = per_w // grp
    assert per_w % grp == 0

    mesh = plsc.VectorSubcoreMesh(core_axis_name="c", subcore_axis_name="s")

    @functools.partial(
        pl.kernel,
        mesh=mesh,
        out_type=jax.ShapeDtypeStruct((LB, HP), jnp.float32),
        scratch_types=[
            pltpu.VMEM((per_w // 128, 128), jnp.int32),
            pltpu.VMEM((grp, HP), jnp.float32),
            pltpu.SemaphoreType.DMA,
        ],
    )
    def gather_k(g_hbm, idx_hbm, out_hbm, idx_v, rows_v, sem):
        wid = lax.axis_index("s") * nc + lax.axis_index("c")
        base = wid * per_w
        pltpu.sync_copy(idx_hbm.at[wid], idx_v)
        for g in range(n_groups):
            off = base + g * grp
            descs = [
                pltpu.async_copy(
                    g_hbm.at[idx_v.at[g * k_grp + j]],
                    rows_v.at[pl.ds(j * 128, 128)],
                    sem,
                )
                for j in range(k_grp)
            ]
            for d in descs:
                d.wait()
            pltpu.sync_copy(rows_v, out_hbm.at[pl.ds(off, grp)])

    return gather_k


def _scan_body(n_steps, t_blk, xb_ref, idx_ref, u_ref, wfc_ref, bfc_ref,
               o_ref, h_ref):
    step = pl.program_id(0)

    @pl.when(step == 0)
    def _init():
        h_ref[...] = jnp.zeros_like(h_ref)

    h = h_ref[...]
    u = u_ref[...]
    for t in range(t_blk):
        x = xb_ref[t]
        m = idx_ref[t] != 0
        hn = jnp.tanh(x + jnp.dot(h, u, preferred_element_type=jnp.float32))
        h = jnp.where(m, hn, h)
    h_ref[...] = h

    @pl.when(step == n_steps - 1)
    def _head():
        logits = (
            jnp.dot(h, wfc_ref[...], preferred_element_type=jnp.float32)
            + bfc_ref[...]
        )
        mx = jnp.max(logits, axis=-1, keepdims=True)
        e = jnp.exp(logits - mx)
        o_ref[...] = e / jnp.sum(e, axis=-1, keepdims=True)


def _rnn_scan(xb3, idx3, Up, Wfcp, bfcp, t_blk=8):
    L, B, _ = xb3.shape
    n_steps = L // t_blk
    return pl.pallas_call(
        functools.partial(_scan_body, n_steps, t_blk),
        grid=(n_steps,),
        in_specs=[
            pl.BlockSpec((t_blk, B, HP), lambda i: (i, 0, 0)),
            pl.BlockSpec((t_blk, B, 1), lambda i: (i, 0, 0)),
            pl.BlockSpec((HP, HP), lambda i: (0, 0)),
            pl.BlockSpec((HP, CP), lambda i: (0, 0)),
            pl.BlockSpec((1, CP), lambda i: (0, 0)),
        ],
        out_specs=pl.BlockSpec((B, CP), lambda i: (0, 0)),
        out_shape=jax.ShapeDtypeStruct((B, CP), jnp.float32),
        scratch_shapes=[pltpu.VMEM((B, HP), jnp.float32)],
    )(xb3, idx3, Up, Wfcp, bfcp)


def kernel(indices, table, W, U, b, Wfc, bfc):
    B, L = indices.shape
    V, E = table.shape
    H = W.shape[1]
    C = Wfc.shape[1]

    Wp = jnp.pad(W, ((0, 0), (0, HP - H)))
    bp = jnp.pad(b, (0, HP - H)).reshape(1, HP)
    Up = jnp.pad(U, ((0, HP - H), (0, HP - H)))
    Wfcp = jnp.pad(Wfc, ((0, HP - H), (0, CP - C)))
    bfcp = jnp.pad(bfc, (0, CP - C), constant_values=-1e30).reshape(1, CP)

    G = _table_times_w(table, Wp, bp)

    idxT = jnp.swapaxes(indices, 0, 1)
    LB = L * B

    info = plsc.get_sparse_core_info()
    nw = info.num_cores * info.num_subcores
    idx2 = idxT.reshape(nw, LB // nw // 128, 128).astype(jnp.int32)
    gather_k = _make_gather(LB, info.num_cores, info.num_subcores)
    xb = gather_k(G, idx2)

    xb3 = xb.reshape(L, B, HP)
    idx3 = idxT.reshape(L, B, 1)
    probs = _rnn_scan(xb3, idx3, Up, Wfcp, bfcp)
    return probs[:, :C]
